# baseline (speedup 1.0000x reference)
import functools
import jax
import jax.numpy as jnp
from jax import lax
from jax.experimental import pallas as pl
from jax.experimental.pallas import tpu as pltpu
from jax.experimental.pallas import tpu_sc as plsc

VOCAB = 100000
ROWS = 512
NC = 2
NS = 16
NW = NC * NS
ROWS_PER_W = ROWS // NW


def _sc_body(pred_hbm, val_hbm, out_hbm, rowbuf, pred_v, val_v, patch_v,
             idx_v, sem, psem):
    c = lax.axis_index("c")
    s = lax.axis_index("s")
    wid = s * NC + c
    base = wid * ROWS_PER_W
    pltpu.sync_copy(pred_hbm.at[pl.ds(base, ROWS_PER_W)], pred_v)
    pltpu.sync_copy(val_hbm.at[pl.ds(base, ROWS_PER_W)], val_v)

    neg50 = jnp.full((16,), -50.0, dtype=jnp.float32)

    def fill(i, carry):
        rowbuf[pl.ds(i * 16, 16)] = neg50
        return carry

    lax.fori_loop(0, VOCAB // 16, fill, 0, unroll=8)

    lanes = lax.iota(jnp.int32, 16)
    patch_v[...] = 50.0 * val_v[...]
    idx_v[...] = (base + lanes) * VOCAB + pred_v[...]

    copies = []
    for j in range(ROWS_PER_W):
        copies.append(pltpu.make_async_copy(
            rowbuf, out_hbm.at[pl.ds((base + j) * VOCAB, VOCAB)], sem))
    for cp in copies:
        cp.start()
    for cp in copies:
        cp.wait()

    patch = pltpu.make_async_copy(patch_v, out_hbm.at[idx_v], psem)
    patch.start()
    patch.wait()


def kernel(input_ids, val):
    B, S = input_ids.shape
    pred = input_ids.reshape(ROWS)
    val1 = val.reshape(ROWS)
    k = functools.partial(
        pl.kernel,
        out_type=jax.ShapeDtypeStruct((ROWS * VOCAB,), jnp.float32),
        mesh=plsc.VectorSubcoreMesh(core_axis_name="c", subcore_axis_name="s"),
        scratch_types=[
            pltpu.VMEM((VOCAB,), jnp.float32),
            pltpu.VMEM((ROWS_PER_W,), jnp.int32),
            pltpu.VMEM((ROWS_PER_W,), jnp.float32),
            pltpu.VMEM((ROWS_PER_W,), jnp.float32),
            pltpu.VMEM((ROWS_PER_W,), jnp.int32),
            pltpu.SemaphoreType.DMA,
            pltpu.SemaphoreType.DMA,
        ],
    )(_sc_body)
    out = k(pred, val1)
    return out.reshape(B, S, VOCAB)


# speedup vs baseline: 5.6018x; 5.6018x over previous
import jax
import jax.numpy as jnp
from jax.experimental import pallas as pl
from jax.experimental.pallas import tpu as pltpu

VOCAB = 100000
ROWS_BLK = 128
V_BLK = 25600


def _onehot_block(pred_ref, val_ref, out_ref):
    v_block = pl.program_id(1)
    pred = pred_ref[:, 0]
    val = val_ref[:, 0]
    iota = jax.lax.broadcasted_iota(jnp.int32, (ROWS_BLK, V_BLK), 1)
    iota = iota + v_block * V_BLK
    out_ref[...] = jnp.where(
        iota == pred[:, None], 50.0 * val[:, None],
        jnp.float32(-50.0))


def kernel(input_ids, val):
    B, S = input_ids.shape
    rows = B * S
    pred = input_ids.reshape(rows, 1)
    val2 = val.reshape(rows, 1)
    n_row_blocks = rows // ROWS_BLK
    n_v_blocks = (VOCAB + V_BLK - 1) // V_BLK
    out = pl.pallas_call(
        _onehot_block,
        grid=(n_row_blocks, n_v_blocks),
        in_specs=[
            pl.BlockSpec((ROWS_BLK, 1), lambda i, j: (i, 0)),
            pl.BlockSpec((ROWS_BLK, 1), lambda i, j: (i, 0)),
        ],
        out_specs=pl.BlockSpec((ROWS_BLK, V_BLK), lambda i, j: (i, j)),
        out_shape=jax.ShapeDtypeStruct((rows, VOCAB), jnp.float32),
        compiler_params=pltpu.CompilerParams(
            dimension_semantics=("parallel", "parallel")),
    )(pred, val2)
    return out.reshape(B, S, VOCAB)
